# baseline (speedup 1.0000x reference)
import jax
import jax.numpy as jnp
from jax import lax
from jax.experimental import pallas as pl
from jax.experimental.pallas import tpu as pltpu
from jax.experimental.pallas import tpu_sc as plsc

_E, _K = 8, 2
_HBLK = 512


def _ltiny_body(x_ref, wg_ref, bg_ref, l8_ref):
    logits = jax.lax.dot_general(
        x_ref[...], wg_ref[...], (((1,), (1,)), ((), ())),
        preferred_element_type=jnp.float32)
    logits = logits + bg_ref[...]
    l8_ref[...] = logits.T


def _route_body(l8_hbm, p1_hbm, p2_hbm, lbuf, p1buf, p2buf):
    wid = lax.axis_index("s") * 2 + lax.axis_index("c")

    @pl.when(wid == 0)
    def _():
        pltpu.sync_copy(l8_hbm, lbuf)
        sl = pl.ds(0, 16)
        v1 = lbuf[0, sl]
        i1 = jnp.zeros((16,), jnp.int32)
        v2 = jnp.full((16,), -1e30, jnp.float32)
        i2 = jnp.zeros((16,), jnp.int32)
        for e in range(1, _E):
            v = lbuf[e, sl]
            ev = jnp.full((16,), e, jnp.int32)
            v2n = jnp.where(v > v2, v, v2)
            i2n = jnp.where(v > v2, ev, i2)
            v2 = jnp.where(v > v1, v1, v2n)
            i2 = jnp.where(v > v1, i1, i2n)
            i1 = jnp.where(v > v1, ev, i1)
            v1 = jnp.where(v > v1, v, v1)
        p1buf[...] = i1
        p2buf[...] = i2
        pltpu.sync_copy(p1buf, p1_hbm)
        pltpu.sync_copy(p2buf, p2_hbm)


def _gelu(v):
    return v * 0.5 * (1.0 + jax.lax.erf(v * 0.7071067811865476))


def _moe_body(p1_ref, p2_ref, x_ref, wg_ref, bg_ref,
              w1a_ref, b1a_ref, w2a_ref, b2a_ref,
              w1b_ref, b1b_ref, w2b_ref, b2b_ref, o_ref,
              sc0_ref, sc1_ref):
    del p1_ref, p2_ref
    h = pl.program_id(2)
    xb = x_ref[...]

    @pl.when(h == 0)
    def _():
        logits = jax.lax.dot_general(
            xb, wg_ref[...], (((1,), (1,)), ((), ())),
            preferred_element_type=jnp.float32)
        logits = logits + bg_ref[...]
        col = jax.lax.broadcasted_iota(jnp.int32, logits.shape, 1)
        v1 = jnp.max(logits, axis=1, keepdims=True)
        i1 = jnp.min(jnp.where(logits == v1, col, _E), axis=1, keepdims=True)
        masked = jnp.where(col == i1, -jnp.inf, logits)
        v2 = jnp.max(masked, axis=1, keepdims=True)
        p = jnp.exp(v2 - v1)
        d = 1.0 + p
        sc0_ref[...] = 1.0 / d
        sc1_ref[...] = p / d

    g0 = sc0_ref[...]
    g1 = sc1_ref[...]

    def expert(w1_ref, b1_ref, w2_ref, g):
        hpre = jax.lax.dot_general(
            xb, w1_ref[0], (((1,), (1,)), ((), ())),
            preferred_element_type=jnp.float32)
        hact = _gelu(hpre + b1_ref[0]) * g
        return jax.lax.dot_general(
            hact, w2_ref[0], (((1,), (1,)), ((), ())),
            preferred_element_type=jnp.float32)

    contrib = (expert(w1a_ref, b1a_ref, w2a_ref, g0)
               + expert(w1b_ref, b1b_ref, w2b_ref, g1))

    @pl.when(h == 0)
    def _():
        o_ref[...] = contrib + g0 * b2a_ref[0] + g1 * b2b_ref[0]

    @pl.when(h != 0)
    def _():
        o_ref[...] = o_ref[...] + contrib


def kernel(x, W1, b1, W2, b2, Wg, bg):
    B, S, D = x.shape
    E, H, _ = W1.shape
    T = B * S
    x2 = x.reshape(T, D)
    bgr = bg.reshape(1, E)

    l8 = pl.pallas_call(
        _ltiny_body,
        grid=(1,),
        in_specs=[
            pl.BlockSpec((128, D), lambda g: (0, 0)),
            pl.BlockSpec((E, D), lambda g: (0, 0)),
            pl.BlockSpec((1, E), lambda g: (0, 0)),
        ],
        out_specs=pl.BlockSpec((E, 128), lambda g: (0, 0)),
        out_shape=jax.ShapeDtypeStruct((E, 128), jnp.float32),
    )(x2, Wg, bgr)

    mesh = plsc.VectorSubcoreMesh(core_axis_name="c", subcore_axis_name="s")
    p1, p2 = pl.kernel(
        _route_body,
        out_type=[
            jax.ShapeDtypeStruct((16,), jnp.int32),
            jax.ShapeDtypeStruct((16,), jnp.int32),
        ],
        scratch_types=[
            pltpu.VMEM((E, 128), jnp.float32),
            pltpu.VMEM((16,), jnp.int32),
            pltpu.VMEM((16,), jnp.int32),
        ],
        mesh=mesh,
    )(l8)

    b1r = b1.reshape(E, 1, H)
    b2r = b2.reshape(E, 1, D)
    NH = H // _HBLK
    NS = 2
    SB = S // NS
    grid_spec = pltpu.PrefetchScalarGridSpec(
        num_scalar_prefetch=2,
        grid=(B, NS, NH),
        in_specs=[
            pl.BlockSpec((SB, D), lambda b, s, h, p1, p2: (b * 2 + s, 0)),
            pl.BlockSpec((E, D), lambda b, s, h, p1, p2: (0, 0)),
            pl.BlockSpec((1, E), lambda b, s, h, p1, p2: (0, 0)),
            pl.BlockSpec((1, _HBLK, D), lambda b, s, h, p1, p2: (p1[b], h, 0)),
            pl.BlockSpec((1, 1, _HBLK), lambda b, s, h, p1, p2: (p1[b], 0, h)),
            pl.BlockSpec((1, D, _HBLK), lambda b, s, h, p1, p2: (p1[b], 0, h)),
            pl.BlockSpec((1, 1, D), lambda b, s, h, p1, p2: (p1[b], 0, 0)),
            pl.BlockSpec((1, _HBLK, D), lambda b, s, h, p1, p2: (p2[b], h, 0)),
            pl.BlockSpec((1, 1, _HBLK), lambda b, s, h, p1, p2: (p2[b], 0, h)),
            pl.BlockSpec((1, D, _HBLK), lambda b, s, h, p1, p2: (p2[b], 0, h)),
            pl.BlockSpec((1, 1, D), lambda b, s, h, p1, p2: (p2[b], 0, 0)),
        ],
        out_specs=pl.BlockSpec((SB, D), lambda b, s, h, p1, p2: (b * 2 + s, 0)),
        scratch_shapes=[
            pltpu.VMEM((SB, 1), jnp.float32),
            pltpu.VMEM((SB, 1), jnp.float32),
        ],
    )
    out = pl.pallas_call(
        _moe_body,
        grid_spec=grid_spec,
        out_shape=jax.ShapeDtypeStruct((T, D), jnp.float32),
        compiler_params=pltpu.CompilerParams(
            dimension_semantics=("parallel", "parallel", "arbitrary")),
    )(p1, p2, x2, Wg, bgr, W1, b1r, W2, b2r, W1, b1r, W2, b2r)
    return out.reshape(B, S, D)
